# baseline (speedup 1.0000x reference)
import functools

import jax
import jax.numpy as jnp
from jax import lax
from jax.experimental import pallas as pl
from jax.experimental.pallas import tpu as pltpu
from jax.experimental.pallas import tpu_sc as plsc

_UNITS = 128
_DENSE_D = 64
_OUT_D = _UNITS + _DENSE_D
_C = 240
_RB = 256
_LANES = 16
_NGRP = _C // _LANES
_NJ = _UNITS // _LANES

_NS = 4
_I_COLS = 0
_I_ROWS = _NS * _C
_I_OFF = 2 * _NS * _C
_I_END = _I_OFF + 96
_F_VALS = 0
_F_B = _NS * _C
_F_END = _F_B + _UNITS


def _sc_body(rows_hbm, cols_hbm, vals_hbm, dense_hbm, w_hbm, b_hbm, off_hbm,
             out_hbm, acc, g_buf, ibuf, fbuf, sem_g0, sem_g1, sem_i0, sem_i1):
    info = plsc.get_sparse_core_info()
    nc, ns = info.num_cores, info.num_subcores
    wid = lax.axis_index("s") * nc + lax.axis_index("c")

    pltpu.sync_copy(off_hbm, ibuf.at[pl.ds(_I_OFF, 96)])
    pltpu.sync_copy(b_hbm, fbuf.at[pl.ds(_F_B, _UNITS)])

    def idx_descrs(slot, g0):
        sem = sem_i0 if slot % 2 == 0 else sem_i1
        return (
            pltpu.make_async_copy(cols_hbm.at[pl.ds(g0, _C)],
                                  ibuf.at[pl.ds(_I_COLS + slot * _C, _C)],
                                  sem),
            pltpu.make_async_copy(rows_hbm.at[pl.ds(g0, _C)],
                                  ibuf.at[pl.ds(_I_ROWS + slot * _C, _C)],
                                  sem),
            pltpu.make_async_copy(vals_hbm.at[pl.ds(g0, _C)],
                                  fbuf.at[pl.ds(_F_VALS + slot * _C, _C)],
                                  sem),
        )

    def gather_descr(slot, sem):
        return pltpu.make_async_copy(
            w_hbm.at[ibuf.at[pl.ds(_I_COLS + slot * _C, _C)]],
            g_buf.at[slot % 2], sem)

    for sb in range(2):
        vb = wid * 2 + sb
        base_row = vb * _RB
        start = ibuf[pl.ds(_I_OFF + vb, _LANES)][0]
        end = ibuf[pl.ds(_I_OFF + vb + 1, _LANES)][0]

        pltpu.sync_copy(
            dense_hbm.at[
                pl.ds(pl.multiple_of(base_row * _DENSE_D // _UNITS, 8),
                      _RB * _DENSE_D // _UNITS)],
            g_buf.at[0, pl.ds(0, _RB * _DENSE_D // _UNITS)])

        @plsc.parallel_loop(0, _RB, step=1, unroll=4)
        def init_body(r):
            for j in range(_NJ):
                sl = pl.ds(j * _LANES, _LANES)
                acc[r, sl] = fbuf[pl.ds(_F_B + j * _LANES, _LANES)]
            r2 = r // 2
            rc = lax.rem(r, 2) * _DENSE_D
            for j in range(_DENSE_D // _LANES):
                acc[r, pl.ds(_UNITS + j * _LANES, _LANES)] = (
                    g_buf[0, r2, pl.ds(rc + j * _LANES, _LANES)])

        cbase = start - lax.rem(start, 8)
        nch = (end - cbase + _C - 1) // _C

        @pl.when(nch > 0)
        def _prologue():
            for d in idx_descrs(0, pl.multiple_of(cbase, 8)):
                d.start()
            for d in idx_descrs(0, pl.multiple_of(cbase, 8)):
                d.wait()
            gather_descr(0, sem_g0).start()

        @pl.when(nch > 1)
        def _prefetch1():
            for d in idx_descrs(1, pl.multiple_of(cbase + _C, 8)):
                d.start()

        def chunk_body(k, _):
            g0 = pl.multiple_of(cbase + k * _C, 8)
            it = lax.rem(k, _NS)

            def step(itv):
                sgn = (sem_g0, sem_g1)
                nxt = (itv + 1) % _NS
                pre = (itv + 2) % _NS

                @pl.when(k + 1 < nch)
                def _():
                    g1 = pl.multiple_of(g0 + _C, 8)
                    for d in idx_descrs(nxt, g1):
                        d.wait()
                    gather_descr(nxt, sgn[nxt % 2]).start()

                @pl.when(k + 2 < nch)
                def _():
                    g2 = pl.multiple_of(g0 + 2 * _C, 8)
                    for d in idx_descrs(pre, g2):
                        d.start()

                gather_descr(itv, sgn[itv % 2]).wait()

            for itv in range(_NS):
                @pl.when(it == itv)
                def _(itv=itv):
                    step(itv)

            lo = jnp.maximum(start - g0, 0)
            hi = jnp.minimum(end - g0, _C)
            b2 = lax.rem(k, 2)

            @plsc.parallel_loop(0, _NGRP, step=1, unroll=3)
            def grp_body(grp):
                gi = pl.multiple_of(grp * _LANES, _LANES)
                lane = gi + lax.broadcasted_iota(jnp.int32, (_LANES,), 0)
                rows_v = ibuf[pl.ds(_I_ROWS + it * _C + gi, _LANES)]
                vals_v = fbuf[pl.ds(_F_VALS + it * _C + gi, _LANES)]
                valid = (lane >= lo) & (lane < hi)
                v_v = jnp.where(valid, vals_v, 0.0)
                r_v = jnp.clip(rows_v - base_row, 0, _RB - 1)
                for t in range(_LANES):
                    r = r_v[t]
                    v = v_v[t]
                    i = gi + t
                    for j in range(_NJ):
                        sl = pl.ds(j * _LANES, _LANES)
                        plsc.addupdate(acc.at[r, sl],
                                       g_buf[b2, i, sl] * v)
            return 0

        lax.fori_loop(0, nch, chunk_body, 0)
        pltpu.sync_copy(
            acc, out_hbm.at[pl.ds(pl.multiple_of(base_row, 8), _RB)])


def kernel(sp_rows, sp_cols, sp_vals, dense_feat, W, b):
    B = dense_feat.shape[0]
    nnz = sp_rows.shape[0]
    rows = sp_rows.astype(jnp.int32)
    cols = sp_cols.astype(jnp.int32)
    vals = sp_vals.astype(jnp.float32)

    rows_p = jnp.concatenate([rows, jnp.zeros((_C,), jnp.int32)])
    cols_p = jnp.concatenate([cols, jnp.zeros((_C,), jnp.int32)])
    vals_p = jnp.concatenate([vals, jnp.zeros((_C,), jnp.float32)])

    nsb = B // _RB
    bounds = jnp.arange(0, B + 1, _RB, dtype=jnp.int32)
    off = jnp.searchsorted(rows, bounds).astype(jnp.int32)
    off = jnp.concatenate([off, jnp.full((96 - (nsb + 1),), nnz, jnp.int32)])

    mesh = plsc.VectorSubcoreMesh(core_axis_name="c", subcore_axis_name="s")
    run = functools.partial(
        pl.kernel,
        mesh=mesh,
        out_type=jax.ShapeDtypeStruct((B, _OUT_D), jnp.float32),
        scratch_types=[
            pltpu.VMEM((_RB, _OUT_D), jnp.float32),
            pltpu.VMEM((2, _C, _UNITS), jnp.float32),
            pltpu.VMEM((_I_END,), jnp.int32),
            pltpu.VMEM((_F_END,), jnp.float32),
            pltpu.SemaphoreType.DMA,
            pltpu.SemaphoreType.DMA,
            pltpu.SemaphoreType.DMA,
            pltpu.SemaphoreType.DMA,
        ],
    )(_sc_body)
    dense_r = dense_feat.reshape(B * _DENSE_D // _UNITS, _UNITS)
    return run(rows_p, cols_p, vals_p, dense_r, W, b, off)


# speedup vs baseline: 1.2412x; 1.2412x over previous
import functools

import jax
import jax.numpy as jnp
from jax import lax
from jax.experimental import pallas as pl
from jax.experimental.pallas import tpu as pltpu
from jax.experimental.pallas import tpu_sc as plsc

_UNITS = 128
_DENSE_D = 64
_OUT_D = _UNITS + _DENSE_D
_C = 240
_RB = 256
_LANES = 16
_NGRP = _C // _LANES
_NJ = _UNITS // _LANES

_NS = 4
_I_COLS = 0
_I_ROWS = _NS * _C
_I_OFF = 2 * _NS * _C
_I_END = _I_OFF + 96
_F_VALS = 0
_F_B = _NS * _C
_F_END = _F_B + _UNITS


def _sc_body(rows_hbm, cols_hbm, vals_hbm, dense_hbm, w_hbm, b_hbm, off_hbm,
             out_hbm, acc, g_buf, ibuf, fbuf, sem_g0, sem_g1, sem_i0, sem_i1):
    info = plsc.get_sparse_core_info()
    nc, ns = info.num_cores, info.num_subcores
    wid = lax.axis_index("s") * nc + lax.axis_index("c")

    pltpu.sync_copy(off_hbm, ibuf.at[pl.ds(_I_OFF, 96)])
    pltpu.sync_copy(b_hbm, fbuf.at[pl.ds(_F_B, _UNITS)])

    def idx_descrs(slot, g0):
        sem = sem_i0 if slot % 2 == 0 else sem_i1
        return (
            pltpu.make_async_copy(cols_hbm.at[pl.ds(g0, _C)],
                                  ibuf.at[pl.ds(_I_COLS + slot * _C, _C)],
                                  sem),
            pltpu.make_async_copy(rows_hbm.at[pl.ds(g0, _C)],
                                  ibuf.at[pl.ds(_I_ROWS + slot * _C, _C)],
                                  sem),
            pltpu.make_async_copy(vals_hbm.at[pl.ds(g0, _C)],
                                  fbuf.at[pl.ds(_F_VALS + slot * _C, _C)],
                                  sem),
        )

    def gather_descr(slot, sem):
        return pltpu.make_async_copy(
            w_hbm.at[ibuf.at[pl.ds(_I_COLS + slot * _C, _C)]],
            g_buf.at[slot % 2], sem)

    for sb in range(2):
        vb = wid * 2 + sb
        base_row = vb * _RB
        start = ibuf[pl.ds(_I_OFF + vb, _LANES)][0]
        end = ibuf[pl.ds(_I_OFF + vb + 1, _LANES)][0]

        pltpu.sync_copy(
            dense_hbm.at[
                pl.ds(pl.multiple_of(base_row * _DENSE_D // _UNITS, 8),
                      _RB * _DENSE_D // _UNITS)],
            g_buf.at[0, pl.ds(0, _RB * _DENSE_D // _UNITS)])

        @plsc.parallel_loop(0, _RB, step=1, unroll=4)
        def init_body(r):
            for j in range(_NJ):
                sl = pl.ds(j * _LANES, _LANES)
                acc[r, sl] = fbuf[pl.ds(_F_B + j * _LANES, _LANES)]
            r2 = r // 2
            rc = lax.rem(r, 2) * _DENSE_D
            for j in range(_DENSE_D // _LANES):
                acc[r, pl.ds(_UNITS + j * _LANES, _LANES)] = (
                    g_buf[0, r2, pl.ds(rc + j * _LANES, _LANES)])

        cbase = start - lax.rem(start, 8)
        nch = (end - cbase + _C - 1) // _C

        @pl.when(nch > 0)
        def _prologue():
            for d in idx_descrs(0, pl.multiple_of(cbase, 8)):
                d.start()
            for d in idx_descrs(0, pl.multiple_of(cbase, 8)):
                d.wait()
            gather_descr(0, sem_g0).start()

        @pl.when(nch > 1)
        def _prefetch1():
            for d in idx_descrs(1, pl.multiple_of(cbase + _C, 8)):
                d.start()

        def chunk_body(k, _):
            g0 = pl.multiple_of(cbase + k * _C, 8)
            it = lax.rem(k, _NS)

            def step(itv):
                sgn = (sem_g0, sem_g1)
                nxt = (itv + 1) % _NS
                pre = (itv + 2) % _NS

                @pl.when(k + 1 < nch)
                def _():
                    g1 = pl.multiple_of(g0 + _C, 8)
                    for d in idx_descrs(nxt, g1):
                        d.wait()
                    gather_descr(nxt, sgn[nxt % 2]).start()

                @pl.when(k + 2 < nch)
                def _():
                    g2 = pl.multiple_of(g0 + 2 * _C, 8)
                    for d in idx_descrs(pre, g2):
                        d.start()

                gather_descr(itv, sgn[itv % 2]).wait()

            for itv in range(_NS):
                @pl.when(it == itv)
                def _(itv=itv):
                    step(itv)

            lo = jnp.maximum(start - g0, 0)
            hi = jnp.minimum(end - g0, _C)
            b2 = lax.rem(k, 2)

            @plsc.parallel_loop(0, _NGRP, step=1, unroll=2)
            def grp_body(grp):
                gi = pl.multiple_of(grp * _LANES, _LANES)
                lane = gi + lax.broadcasted_iota(jnp.int32, (_LANES,), 0)
                rows_v = ibuf[pl.ds(_I_ROWS + it * _C + gi, _LANES)]
                vals_v = fbuf[pl.ds(_F_VALS + it * _C + gi, _LANES)]
                valid = (lane >= lo) & (lane < hi)
                v_v = jnp.where(valid, vals_v, 0.0)
                r_v = jnp.clip(rows_v - base_row, 0, _RB - 1)
                for t in range(_LANES):
                    r = r_v[t]
                    v = v_v[t]
                    i = gi + t
                    for j in range(_NJ):
                        sl = pl.ds(j * _LANES, _LANES)
                        plsc.addupdate(acc.at[r, sl],
                                       g_buf[b2, i, sl] * v)
            return 0

        lax.fori_loop(0, nch, chunk_body, 0)
        pltpu.sync_copy(
            acc, out_hbm.at[pl.ds(pl.multiple_of(base_row, 8), _RB)])


def kernel(sp_rows, sp_cols, sp_vals, dense_feat, W, b):
    B = dense_feat.shape[0]
    nnz = sp_rows.shape[0]
    rows = sp_rows.astype(jnp.int32)
    cols = sp_cols.astype(jnp.int32)
    vals = sp_vals.astype(jnp.float32)

    rows_p = jnp.concatenate([rows, jnp.zeros((_C,), jnp.int32)])
    cols_p = jnp.concatenate([cols, jnp.zeros((_C,), jnp.int32)])
    vals_p = jnp.concatenate([vals, jnp.zeros((_C,), jnp.float32)])

    nsb = B // _RB
    bounds = jnp.arange(0, B + 1, _RB, dtype=jnp.int32)
    off = jnp.searchsorted(rows, bounds).astype(jnp.int32)
    off = jnp.concatenate([off, jnp.full((96 - (nsb + 1),), nnz, jnp.int32)])

    mesh = plsc.VectorSubcoreMesh(core_axis_name="c", subcore_axis_name="s")
    run = functools.partial(
        pl.kernel,
        mesh=mesh,
        out_type=jax.ShapeDtypeStruct((B, _OUT_D), jnp.float32),
        scratch_types=[
            pltpu.VMEM((_RB, _OUT_D), jnp.float32),
            pltpu.VMEM((2, _C, _UNITS), jnp.float32),
            pltpu.VMEM((_I_END,), jnp.int32),
            pltpu.VMEM((_F_END,), jnp.float32),
            pltpu.SemaphoreType.DMA,
            pltpu.SemaphoreType.DMA,
            pltpu.SemaphoreType.DMA,
            pltpu.SemaphoreType.DMA,
        ],
    )(_sc_body)
    dense_r = dense_feat.reshape(B * _DENSE_D // _UNITS, _UNITS)
    return run(rows_p, cols_p, vals_p, dense_r, W, b, off)
